# baseline (speedup 1.0000x reference)
import functools

import jax
import jax.numpy as jnp
from jax import lax
from jax.experimental import pallas as pl
from jax.experimental.pallas import tpu as pltpu
from jax.experimental.pallas import tpu_sc as plsc

_N_DATA = 1000000
_EMBED_DIM = 32
_HIDDENS = 128
_OUTPUT_DIM = 64
_BATCH = 16384
_LANES = 128

_NC = 2
_NS = 16
_NW = _NC * _NS
_B_PER_W = _BATCH // _NW
_WSZ = 8
_N_WIN = _B_PER_W // _WSZ


def _make_sc_gather():
    mesh = plsc.VectorSubcoreMesh(core_axis_name="c", subcore_axis_name="s")

    @functools.partial(
        pl.kernel,
        mesh=mesh,
        compiler_params=pltpu.CompilerParams(needs_layout_passes=False),
        out_type=jax.ShapeDtypeStruct((_BATCH * _EMBED_DIM,), jnp.float32),
        scratch_types=[
            pltpu.VMEM((_B_PER_W + 16,), jnp.int32),
            pltpu.VMEM((2, _WSZ, _EMBED_DIM, _LANES), jnp.float32),
            pltpu.VMEM((_B_PER_W * _EMBED_DIM,), jnp.float32),
            pltpu.SemaphoreType.DMA,
        ],
    )
    def gather_kernel(tableT_hbm, idx_hbm, out_hbm, idx_v, blk_v, rows_v, sem):
        wid = lax.axis_index("s") * _NC + lax.axis_index("c")
        base = wid * _B_PER_W
        pltpu.sync_copy(idx_hbm.at[pl.ds(base, _B_PER_W)],
                        idx_v.at[pl.ds(0, _B_PER_W)])
        lane = lax.iota(jnp.int32, 16)

        def _issue(w, b):
            v = idx_v[pl.ds(w * _WSZ, 16)]
            for l in range(_WSZ):
                gt = v[l] >> 7
                off = pl.multiple_of(gt * _LANES, _LANES)
                for q in range(4):
                    pltpu.async_copy(
                        tableT_hbm.at[pl.ds(q * 8, 8), pl.ds(off, _LANES)],
                        blk_v.at[b, l, pl.ds(q * 8, 8)], sem)

        def _drain_extract(w, b):
            v = idx_v[pl.ds(w * _WSZ, 16)]
            bv = jnp.full((16,), b, jnp.int32)
            for l in range(_WSZ):
                for q in range(4):
                    pltpu.make_async_copy(
                        tableT_hbm.at[pl.ds(q * 8, 8), pl.ds(0, _LANES)],
                        blk_v.at[b, l, pl.ds(q * 8, 8)], sem).wait()
                gr = jnp.full((16,), v[l] & (_LANES - 1), jnp.int32)
                sv = jnp.full((16,), l, jnp.int32)
                lo = plsc.load_gather(blk_v, [bv, sv, lane, gr])
                hi = plsc.load_gather(blk_v, [bv, sv, lane + 16, gr])
                rows_v[pl.ds((w * _WSZ + l) * _EMBED_DIM, 16)] = lo
                rows_v[pl.ds((w * _WSZ + l) * _EMBED_DIM + 16, 16)] = hi

        @pl.loop(0, _N_WIN)
        def _win(w):
            b = lax.rem(w, 2)

            @pl.when(w == 0)
            def _():
                _issue(w, b)

            @pl.when(w + 1 < _N_WIN)
            def _():
                _issue(w + 1, 1 - b)

            _drain_extract(w, b)

        pltpu.sync_copy(
            rows_v, out_hbm.at[pl.ds(base * _EMBED_DIM, _B_PER_W * _EMBED_DIM)])

    return gather_kernel


_sc_gather = _make_sc_gather()


_PACK = 4
_ROWS4 = _BATCH // _PACK


def _mlp_body(x_ref, w1_ref, b1_ref, w2T_ref, b2_ref, oT_ref):
    blk = x_ref.shape[0]
    h4 = jnp.tanh(
        jnp.dot(x_ref[...], w1_ref[...], preferred_element_type=jnp.float32)
        + b1_ref[...])
    h = h4.reshape(blk * _PACK, _HIDDENS)
    oT_ref[...] = jax.nn.sigmoid(
        lax.dot_general(w2T_ref[...], h, (((1,), (1,)), ((), ())),
                        preferred_element_type=jnp.float32)
        + b2_ref[...])


def _mlp(emb4, W1big, b1big, W2T, b2col):
    blk = 2048
    grid = (_ROWS4 // blk,)
    return pl.pallas_call(
        _mlp_body,
        grid=grid,
        in_specs=[
            pl.BlockSpec((blk, _PACK * _EMBED_DIM), lambda i: (i, 0)),
            pl.BlockSpec((_PACK * _EMBED_DIM, _PACK * _HIDDENS),
                         lambda i: (0, 0)),
            pl.BlockSpec((1, _PACK * _HIDDENS), lambda i: (0, 0)),
            pl.BlockSpec((_OUTPUT_DIM, _HIDDENS), lambda i: (0, 0)),
            pl.BlockSpec((_OUTPUT_DIM, 1), lambda i: (0, 0)),
        ],
        out_specs=pl.BlockSpec((_OUTPUT_DIM, blk * _PACK), lambda i: (0, i)),
        out_shape=jax.ShapeDtypeStruct((_OUTPUT_DIM, _BATCH), jnp.float32),
    )(emb4, W1big, b1big, W2T, b2col)


def _block_diag(W, k):
    m, n = W.shape
    out = W[None, :, None, :] * jnp.eye(k, dtype=W.dtype)[:, None, :, None]
    return out.reshape(k * m, k * n)


def kernel(i, table, W1, b1, W2, b2):
    emb4 = _sc_gather(table.T, i.astype(jnp.int32)).reshape(
        _ROWS4, _PACK * _EMBED_DIM)
    W1big = _block_diag(W1, _PACK)
    b1big = jnp.tile(b1, _PACK).reshape(1, _PACK * _HIDDENS)
    outT = _mlp(emb4, W1big, b1big, W2.T, b2.reshape(_OUTPUT_DIM, 1))
    return outT.T


# speedup vs baseline: 1.0106x; 1.0106x over previous
import functools

import jax
import jax.numpy as jnp
from jax import lax
from jax.experimental import pallas as pl
from jax.experimental.pallas import tpu as pltpu
from jax.experimental.pallas import tpu_sc as plsc

_N_DATA = 1000000
_EMBED_DIM = 32
_HIDDENS = 128
_OUTPUT_DIM = 64
_BATCH = 16384
_LANES = 128

_NC = 2
_NS = 16
_NW = _NC * _NS
_B_PER_W = _BATCH // _NW
_WSZ = 8
_N_WIN = _B_PER_W // _WSZ


def _make_sc_gather():
    mesh = plsc.VectorSubcoreMesh(core_axis_name="c", subcore_axis_name="s")

    @functools.partial(
        pl.kernel,
        mesh=mesh,
        compiler_params=pltpu.CompilerParams(needs_layout_passes=False),
        out_type=jax.ShapeDtypeStruct((_BATCH * _EMBED_DIM,), jnp.float32),
        scratch_types=[
            pltpu.VMEM((_B_PER_W + 16,), jnp.int32),
            pltpu.VMEM((2, _WSZ, _EMBED_DIM, _LANES), jnp.float32),
            pltpu.VMEM((_B_PER_W * _EMBED_DIM,), jnp.float32),
            pltpu.SemaphoreType.DMA,
        ],
    )
    def gather_kernel(tableT_hbm, idx_hbm, out_hbm, idx_v, blk_v, rows_v, sem):
        wid = lax.axis_index("s") * _NC + lax.axis_index("c")
        base = wid * _B_PER_W
        pltpu.sync_copy(idx_hbm.at[pl.ds(base, _B_PER_W)],
                        idx_v.at[pl.ds(0, _B_PER_W)])
        lane = lax.iota(jnp.int32, 16)

        def _issue(w, b):
            v = idx_v[pl.ds(w * _WSZ, 16)]
            for l in range(_WSZ):
                gt = v[l] >> 7
                pltpu.async_copy(
                    tableT_hbm.at[:, pl.ds(pl.multiple_of(gt * _LANES, _LANES),
                                           _LANES)],
                    blk_v.at[b, l], sem)

        def _drain_extract(w, b):
            v = idx_v[pl.ds(w * _WSZ, 16)]
            bv = jnp.full((16,), b, jnp.int32)
            for l in range(_WSZ):
                pltpu.make_async_copy(
                    tableT_hbm.at[:, pl.ds(0, _LANES)],
                    blk_v.at[b, l], sem).wait()
                gr = jnp.full((16,), v[l] & (_LANES - 1), jnp.int32)
                sv = jnp.full((16,), l, jnp.int32)
                lo = plsc.load_gather(blk_v, [bv, sv, lane, gr])
                hi = plsc.load_gather(blk_v, [bv, sv, lane + 16, gr])
                rows_v[pl.ds((w * _WSZ + l) * _EMBED_DIM, 16)] = lo
                rows_v[pl.ds((w * _WSZ + l) * _EMBED_DIM + 16, 16)] = hi

        @pl.loop(0, _N_WIN)
        def _win(w):
            b = lax.rem(w, 2)

            @pl.when(w == 0)
            def _():
                _issue(w, b)

            @pl.when(w + 1 < _N_WIN)
            def _():
                _issue(w + 1, 1 - b)

            _drain_extract(w, b)

        pltpu.sync_copy(
            rows_v, out_hbm.at[pl.ds(base * _EMBED_DIM, _B_PER_W * _EMBED_DIM)])

    return gather_kernel


_sc_gather = _make_sc_gather()


_PACK = 4
_ROWS4 = _BATCH // _PACK


def _mlp_body(x_ref, w1_ref, b1_ref, w2T_ref, b2_ref, oT_ref):
    blk = x_ref.shape[0]
    h4 = jnp.tanh(
        jnp.dot(x_ref[...], w1_ref[...], preferred_element_type=jnp.float32)
        + b1_ref[...])
    h = h4.reshape(blk * _PACK, _HIDDENS)
    oT_ref[...] = jax.nn.sigmoid(
        lax.dot_general(w2T_ref[...], h, (((1,), (1,)), ((), ())),
                        preferred_element_type=jnp.float32)
        + b2_ref[...])


def _mlp(emb4, W1big, b1big, W2T, b2col):
    blk = 2048
    grid = (_ROWS4 // blk,)
    return pl.pallas_call(
        _mlp_body,
        grid=grid,
        in_specs=[
            pl.BlockSpec((blk, _PACK * _EMBED_DIM), lambda i: (i, 0)),
            pl.BlockSpec((_PACK * _EMBED_DIM, _PACK * _HIDDENS),
                         lambda i: (0, 0)),
            pl.BlockSpec((1, _PACK * _HIDDENS), lambda i: (0, 0)),
            pl.BlockSpec((_OUTPUT_DIM, _HIDDENS), lambda i: (0, 0)),
            pl.BlockSpec((_OUTPUT_DIM, 1), lambda i: (0, 0)),
        ],
        out_specs=pl.BlockSpec((_OUTPUT_DIM, blk * _PACK), lambda i: (0, i)),
        out_shape=jax.ShapeDtypeStruct((_OUTPUT_DIM, _BATCH), jnp.float32),
    )(emb4, W1big, b1big, W2T, b2col)


def _block_diag(W, k):
    m, n = W.shape
    out = W[None, :, None, :] * jnp.eye(k, dtype=W.dtype)[:, None, :, None]
    return out.reshape(k * m, k * n)


def kernel(i, table, W1, b1, W2, b2):
    emb4 = _sc_gather(table.T, i.astype(jnp.int32)).reshape(
        _ROWS4, _PACK * _EMBED_DIM)
    W1big = _block_diag(W1, _PACK)
    b1big = jnp.tile(b1, _PACK).reshape(1, _PACK * _HIDDENS)
    outT = _mlp(emb4, W1big, b1big, W2.T, b2.reshape(_OUTPUT_DIM, 1))
    return outT.T
